# baseline (speedup 1.0000x reference)
import functools
import math

import jax
import jax.numpy as jnp
from jax import lax
from jax.experimental import pallas as pl
from jax.experimental.pallas import tpu as pltpu
from jax.experimental.pallas import tpu_sc as plsc

D_MODEL = 128
SCALE = math.sqrt(float(D_MODEL))
NUM_CORES = 2
NUM_SUBCORES = 16
NW = NUM_CORES * NUM_SUBCORES
N_I = 4096
N_J = 50
B_TOTAL = N_I * N_J
BPW = B_TOTAL // NW
CHUNK = 128
NCHUNK = BPW // CHUNK
NBUF = 6
MAIN = (NCHUNK // NBUF) * NBUF
LANES = 16
VECS_PER_ROW = D_MODEL // LANES

_mesh = plsc.VectorSubcoreMesh(core_axis_name="c", subcore_axis_name="s")


@functools.partial(
    pl.kernel,
    out_type=jax.ShapeDtypeStruct((B_TOTAL, D_MODEL), jnp.float32),
    mesh=_mesh,
    compiler_params=pltpu.CompilerParams(use_tc_tiling_on_sc=True),
    scratch_types=[
        pltpu.VMEM((BPW,), jnp.int32),
    ] + [pltpu.VMEM((CHUNK, D_MODEL), jnp.float32)] * 6
      + [pltpu.SemaphoreType.DMA] * 12,
)
def _emb_lookup(idx_hbm, table_hbm, out_hbm, idx_v, buf0, buf1, buf2, buf3,
                buf4, buf5, in0, in1, in2, in3, in4, in5,
                ot0, ot1, ot2, ot3, ot4, ot5):
    wid = lax.axis_index("s") * NUM_CORES + lax.axis_index("c")
    base = wid * BPW
    pltpu.sync_copy(idx_hbm.at[pl.ds(base, BPW)], idx_v)

    bufs = (buf0, buf1, buf2, buf3, buf4, buf5)
    in_sems = (in0, in1, in2, in3, in4, in5)
    out_sems = (ot0, ot1, ot2, ot3, ot4, ot5)

    def start_gather(g, b):
        pltpu.async_copy(
            table_hbm.at[idx_v.at[pl.ds(g * CHUNK, CHUNK)]], bufs[b], in_sems[b]
        )

    def wait_gather(g, b):
        pltpu.make_async_copy(
            table_hbm.at[idx_v.at[pl.ds(g * CHUNK, CHUNK)]], bufs[b], in_sems[b]
        ).wait()

    def start_store(g, b):
        pltpu.async_copy(
            bufs[b], out_hbm.at[pl.ds(base + g * CHUNK, CHUNK)], out_sems[b]
        )

    def wait_store(g, b):
        pltpu.make_async_copy(
            bufs[b], out_hbm.at[pl.ds(base + g * CHUNK, CHUNK)], out_sems[b]
        ).wait()

    def scale(b):
        buf = bufs[b]

        @plsc.parallel_loop(0, CHUNK, unroll=4)
        def _scale(i):
            for j in range(VECS_PER_ROW):
                sl = pl.ds(j * LANES, LANES)
                buf[i, sl] = buf[i, sl] * SCALE

    def body(g, b):
        @pl.when(g + 5 < NCHUNK)
        def _prefetch():
            pb = (b + 5) % NBUF

            @pl.when(g >= 1)
            def _drain():
                wait_store(g - 1, pb)

            start_gather(g + 5, pb)

        wait_gather(g, b)
        scale(b)
        start_store(g, b)

    for _g in range(5):
        start_gather(_g, _g)

    @pl.loop(0, MAIN, step=NBUF)
    def _outer(g0):
        for b in range(NBUF):
            body(g0 + b, b)

    for g in range(MAIN, NCHUNK):
        body(g, g % NBUF)

    for g in range(NCHUNK - NBUF, NCHUNK):
        wait_store(g, g % NBUF)


def kernel(x, lut):
    idx = x.T.reshape(-1).astype(jnp.int32)
    out = _emb_lookup(idx, lut)
    return out.reshape(N_J, N_I, D_MODEL).transpose(1, 0, 2)
